# baseline (speedup 1.0000x reference)
import functools

import jax
import jax.numpy as jnp
from jax.experimental import pallas as pl
from jax.experimental.pallas import tpu as pltpu

_B = 32768
_D_DOR = 128
_D_VEN = 128
_N_EXP = 8
_D_EXP = 64
_D_OUT = 64
_FB = 0.5


def _moe_kernel(d_ref, v_ref, gwd_ref, gwv_ref, w1d_ref, w1v_ref, b1_ref,
                w2s_ref, b2_ref, woT_ref, bo_ref, wfdT_ref, wfvT_ref, exp_ref,
                assoc_ref, fbd_ref, fbv_ref, gw_ref):
    d = d_ref[...]
    v = v_ref[...]
    f32 = jnp.float32

    logits = (jnp.dot(d, gwd_ref[...], preferred_element_type=f32)
              + jnp.dot(v, gwv_ref[...], preferred_element_type=f32))

    iota = jax.lax.broadcasted_iota(jnp.int32, logits.shape, 1)
    m1 = jnp.max(logits, axis=-1, keepdims=True)
    eq1 = logits == m1
    i1 = jnp.min(jnp.where(eq1, iota, _N_EXP), axis=-1, keepdims=True)
    one1 = iota == i1
    l2 = jnp.where(one1, -jnp.inf, logits)
    m2 = jnp.max(l2, axis=-1, keepdims=True)
    eq2 = l2 == m2
    i2 = jnp.min(jnp.where(eq2, iota, _N_EXP), axis=-1, keepdims=True)
    keep = one1 | (iota == i2)

    e = jnp.where(keep, jnp.exp(logits - m1), 0.0)
    gw = e / jnp.sum(e, axis=-1, keepdims=True)
    gw_ref[...] = gw

    bf16 = jnp.bfloat16
    d16 = d.astype(bf16)
    v16 = v.astype(bf16)
    h = (jnp.dot(d16, w1d_ref[...], preferred_element_type=f32)
         + jnp.dot(v16, w1v_ref[...], preferred_element_type=f32)
         + b1_ref[...])
    h = 0.5 * h * (1.0 + jax.lax.erf(h * jnp.float32(0.7071067811865476)))

    gwx = jnp.dot(gw, exp_ref[...], preferred_element_type=f32)
    hs = (h * gwx).astype(bf16)
    bound = (jnp.dot(hs, w2s_ref[...], preferred_element_type=f32)
             + jnp.dot(gw, b2_ref[...], preferred_element_type=f32))

    assoc = jnp.dot(bound.astype(bf16), woT_ref[...], preferred_element_type=f32) + bo_ref[...]
    assoc_ref[...] = assoc
    a16 = assoc.astype(bf16)
    fbd_ref[...] = _FB * jnp.dot(a16, wfdT_ref[...], preferred_element_type=f32)
    fbv_ref[...] = _FB * jnp.dot(a16, wfvT_ref[...], preferred_element_type=f32)


@functools.partial(jax.jit, static_argnames=("block",))
def _run(dorsal, ventral, gate_w, w1, b1, w2, b2, wo, bo, wfd, wfv, block=1024):
    gwT = gate_w.T
    gwd, gwv = gwT[:_D_DOR], gwT[_D_DOR:]
    w1cat = w1.transpose(2, 0, 1).reshape(_D_DOR + _D_VEN, _N_EXP * _D_EXP)
    w1cat = w1cat.astype(jnp.bfloat16)
    w1d, w1v = w1cat[:_D_DOR], w1cat[_D_DOR:]
    b1row = b1.reshape(1, _N_EXP * _D_EXP)
    w2s = w2.transpose(0, 2, 1).reshape(_N_EXP * _D_EXP, _D_EXP).astype(jnp.bfloat16)
    woT = wo.T.astype(jnp.bfloat16)
    borow = bo.reshape(1, _D_OUT)
    wfdT = wfd.T.astype(jnp.bfloat16)
    wfvT = wfv.T.astype(jnp.bfloat16)
    expand = jnp.repeat(jnp.eye(_N_EXP, dtype=jnp.float32), _D_EXP, axis=1)

    grid = (_B // block,)
    tok = lambda i: (i, 0)
    full = lambda i: (0, 0)
    out_shapes = (
        jax.ShapeDtypeStruct((_B, _D_OUT), jnp.float32),
        jax.ShapeDtypeStruct((_B, _D_DOR), jnp.float32),
        jax.ShapeDtypeStruct((_B, _D_VEN), jnp.float32),
        jax.ShapeDtypeStruct((_B, _N_EXP), jnp.float32),
    )
    return pl.pallas_call(
        _moe_kernel,
        grid=grid,
        in_specs=[
            pl.BlockSpec((block, _D_DOR), tok),
            pl.BlockSpec((block, _D_VEN), tok),
            pl.BlockSpec((_D_DOR, _N_EXP), full),
            pl.BlockSpec((_D_VEN, _N_EXP), full),
            pl.BlockSpec((_D_DOR, _N_EXP * _D_EXP), full),
            pl.BlockSpec((_D_VEN, _N_EXP * _D_EXP), full),
            pl.BlockSpec((1, _N_EXP * _D_EXP), full),
            pl.BlockSpec((_N_EXP * _D_EXP, _D_EXP), full),
            pl.BlockSpec((_N_EXP, _D_EXP), full),
            pl.BlockSpec((_D_EXP, _D_OUT), full),
            pl.BlockSpec((1, _D_OUT), full),
            pl.BlockSpec((_D_OUT, _D_DOR), full),
            pl.BlockSpec((_D_OUT, _D_VEN), full),
            pl.BlockSpec((_N_EXP, _N_EXP * _D_EXP), full),
        ],
        out_specs=(
            pl.BlockSpec((block, _D_OUT), tok),
            pl.BlockSpec((block, _D_DOR), tok),
            pl.BlockSpec((block, _D_VEN), tok),
            pl.BlockSpec((block, _N_EXP), tok),
        ),
        out_shape=out_shapes,
        compiler_params=pltpu.CompilerParams(
            dimension_semantics=("arbitrary",),
        ),
    )(dorsal, ventral, gwd, gwv, w1d, w1v, b1row, w2s, b2, woT, borow,
      wfdT, wfvT, expand)


def kernel(dorsal, ventral, gate_w, w1, b1, w2, b2, wo, bo, wfd, wfv):
    return _run(dorsal, ventral, gate_w, w1, b1, w2, b2, wo, bo, wfd, wfv)


# speedup vs baseline: 1.2647x; 1.2647x over previous
import functools

import jax
import jax.numpy as jnp
from jax.experimental import pallas as pl
from jax.experimental.pallas import tpu as pltpu

_B = 32768
_D_DOR = 128
_D_VEN = 128
_N_EXP = 8
_D_EXP = 64
_D_OUT = 64
_FB = 0.5


def _moe_kernel(d_ref, v_ref, gwd_ref, gwv_ref, w1d_ref, w1v_ref, b1_ref,
                w2o_ref, b2o_ref, bo_ref, wf_ref, exp_ref,
                assoc_ref, fbd_ref, fbv_ref, gw_ref):
    d = d_ref[...]
    v = v_ref[...]
    f32 = jnp.float32

    logits = (jnp.dot(d, gwd_ref[...], preferred_element_type=f32)
              + jnp.dot(v, gwv_ref[...], preferred_element_type=f32))

    m1 = jnp.max(logits, axis=-1, keepdims=True)
    l2 = jnp.where(logits == m1, jnp.float32(-1e30), logits)
    m2 = jnp.max(l2, axis=-1, keepdims=True)
    keep = logits >= m2

    e = jnp.where(keep, jnp.exp(logits - m1), 0.0)
    gw = e / jnp.sum(e, axis=-1, keepdims=True)
    gw_ref[...] = gw

    bf16 = jnp.bfloat16
    d16 = d.astype(bf16)
    v16 = v.astype(bf16)
    h = (jnp.dot(d16, w1d_ref[...], preferred_element_type=f32)
         + jnp.dot(v16, w1v_ref[...], preferred_element_type=f32)
         + b1_ref[...])
    h = 0.5 * h * (1.0 + jax.lax.erf(h * jnp.float32(0.7071067811865476)))

    gwx = jnp.dot(gw, exp_ref[...], preferred_element_type=f32)
    hs = (h * gwx).astype(bf16)
    assoc = (jnp.dot(hs, w2o_ref[...], preferred_element_type=f32)
             + jnp.dot(gw, b2o_ref[...], preferred_element_type=f32)
             + bo_ref[...])
    assoc_ref[...] = assoc
    fb = _FB * jnp.dot(assoc.astype(bf16), wf_ref[...], preferred_element_type=f32)
    fbd_ref[...] = fb[:, :_D_DOR]
    fbv_ref[...] = fb[:, _D_DOR:]


@functools.partial(jax.jit, static_argnames=("block",))
def _run(dorsal, ventral, gate_w, w1, b1, w2, b2, wo, bo, wfd, wfv, block=1024):
    gwT = gate_w.T
    gwd, gwv = gwT[:_D_DOR], gwT[_D_DOR:]
    w1cat = w1.transpose(2, 0, 1).reshape(_D_DOR + _D_VEN, _N_EXP * _D_EXP)
    w1cat = w1cat.astype(jnp.bfloat16)
    w1d, w1v = w1cat[:_D_DOR], w1cat[_D_DOR:]
    b1row = b1.reshape(1, _N_EXP * _D_EXP)
    w2s = w2.transpose(0, 2, 1).reshape(_N_EXP * _D_EXP, _D_EXP)
    w2o = (w2s @ wo.T).astype(jnp.bfloat16)
    b2o = b2 @ wo.T
    borow = bo.reshape(1, _D_OUT)
    wf = jnp.concatenate([wfd.T, wfv.T], axis=1).astype(jnp.bfloat16)
    expand = jnp.repeat(jnp.eye(_N_EXP, dtype=jnp.float32), _D_EXP, axis=1)

    grid = (_B // block,)
    tok = lambda i: (i, 0)
    full = lambda i: (0, 0)
    out_shapes = (
        jax.ShapeDtypeStruct((_B, _D_OUT), jnp.float32),
        jax.ShapeDtypeStruct((_B, _D_DOR), jnp.float32),
        jax.ShapeDtypeStruct((_B, _D_VEN), jnp.float32),
        jax.ShapeDtypeStruct((_B, _N_EXP), jnp.float32),
    )
    return pl.pallas_call(
        _moe_kernel,
        grid=grid,
        in_specs=[
            pl.BlockSpec((block, _D_DOR), tok),
            pl.BlockSpec((block, _D_VEN), tok),
            pl.BlockSpec((_D_DOR, _N_EXP), full),
            pl.BlockSpec((_D_VEN, _N_EXP), full),
            pl.BlockSpec((_D_DOR, _N_EXP * _D_EXP), full),
            pl.BlockSpec((_D_VEN, _N_EXP * _D_EXP), full),
            pl.BlockSpec((1, _N_EXP * _D_EXP), full),
            pl.BlockSpec((_N_EXP * _D_EXP, _D_OUT), full),
            pl.BlockSpec((_N_EXP, _D_OUT), full),
            pl.BlockSpec((1, _D_OUT), full),
            pl.BlockSpec((_D_EXP, _D_DOR + _D_VEN), full),
            pl.BlockSpec((_N_EXP, _N_EXP * _D_EXP), full),
        ],
        out_specs=(
            pl.BlockSpec((block, _D_OUT), tok),
            pl.BlockSpec((block, _D_DOR), tok),
            pl.BlockSpec((block, _D_VEN), tok),
            pl.BlockSpec((block, _N_EXP), tok),
        ),
        out_shape=out_shapes,
        compiler_params=pltpu.CompilerParams(
            dimension_semantics=("arbitrary",),
        ),
    )(dorsal, ventral, gwd, gwv, w1d, w1v, b1row, w2o, b2o, borow, wf, expand)


def kernel(dorsal, ventral, gate_w, w1, b1, w2, b2, wo, bo, wfd, wfv):
    return _run(dorsal, ventral, gate_w, w1, b1, w2, b2, wo, bo, wfd, wfv)


# speedup vs baseline: 1.2954x; 1.0242x over previous
import functools

import jax
import jax.numpy as jnp
from jax.experimental import pallas as pl
from jax.experimental.pallas import tpu as pltpu

_B = 32768
_D_DOR = 128
_D_VEN = 128
_N_EXP = 8
_D_EXP = 64
_D_OUT = 64
_FB = 0.5


def _moe_kernel(d_ref, v_ref, gwd_ref, gwv_ref, w1d_ref, w1v_ref, b1_ref,
                w2o_ref, b2o_ref, bo_ref, wf_ref, exp_ref,
                assoc_ref, fbd_ref, fbv_ref, gw_ref):
    d = d_ref[...]
    v = v_ref[...]
    f32 = jnp.float32

    logits = (jnp.dot(d, gwd_ref[...], preferred_element_type=f32)
              + jnp.dot(v, gwv_ref[...], preferred_element_type=f32))

    m1 = jnp.max(logits, axis=-1, keepdims=True)
    l2 = jnp.where(logits == m1, jnp.float32(-1e30), logits)
    m2 = jnp.max(l2, axis=-1, keepdims=True)
    keep = logits >= m2

    e = jnp.where(keep, jnp.exp(logits - m1), 0.0)
    gw = e / jnp.sum(e, axis=-1, keepdims=True)
    gw_ref[...] = gw

    bf16 = jnp.bfloat16
    d16 = d.astype(bf16)
    v16 = v.astype(bf16)
    h = (jnp.dot(d16, w1d_ref[...], preferred_element_type=f32)
         + jnp.dot(v16, w1v_ref[...], preferred_element_type=f32)
         + b1_ref[...])
    h = 0.5 * h * (1.0 + jax.lax.erf(h * jnp.float32(0.7071067811865476)))

    gwx = jnp.dot(gw, exp_ref[...], preferred_element_type=f32)
    hs = (h * gwx).astype(bf16)
    assoc = (jnp.dot(hs, w2o_ref[...], preferred_element_type=f32)
             + jnp.dot(gw, b2o_ref[...], preferred_element_type=f32)
             + bo_ref[...])
    assoc_ref[...] = assoc
    fb = _FB * jnp.dot(assoc.astype(bf16), wf_ref[...], preferred_element_type=f32)
    fbd_ref[...] = fb[:, :_D_DOR]
    fbv_ref[...] = fb[:, _D_DOR:]


@functools.partial(jax.jit, static_argnames=("block",))
def _run(dorsal, ventral, gate_w, w1, b1, w2, b2, wo, bo, wfd, wfv, block=2048):
    gwT = gate_w.T
    gwd, gwv = gwT[:_D_DOR], gwT[_D_DOR:]
    w1cat = w1.transpose(2, 0, 1).reshape(_D_DOR + _D_VEN, _N_EXP * _D_EXP)
    w1cat = w1cat.astype(jnp.bfloat16)
    w1d, w1v = w1cat[:_D_DOR], w1cat[_D_DOR:]
    b1row = b1.reshape(1, _N_EXP * _D_EXP)
    w2s = w2.transpose(0, 2, 1).reshape(_N_EXP * _D_EXP, _D_EXP)
    w2o = (w2s @ wo.T).astype(jnp.bfloat16)
    b2o = b2 @ wo.T
    borow = bo.reshape(1, _D_OUT)
    wf = jnp.concatenate([wfd.T, wfv.T], axis=1).astype(jnp.bfloat16)
    expand = jnp.repeat(jnp.eye(_N_EXP, dtype=jnp.float32), _D_EXP, axis=1)

    grid = (_B // block,)
    tok = lambda i: (i, 0)
    full = lambda i: (0, 0)
    out_shapes = (
        jax.ShapeDtypeStruct((_B, _D_OUT), jnp.float32),
        jax.ShapeDtypeStruct((_B, _D_DOR), jnp.float32),
        jax.ShapeDtypeStruct((_B, _D_VEN), jnp.float32),
        jax.ShapeDtypeStruct((_B, _N_EXP), jnp.float32),
    )
    return pl.pallas_call(
        _moe_kernel,
        grid=grid,
        in_specs=[
            pl.BlockSpec((block, _D_DOR), tok),
            pl.BlockSpec((block, _D_VEN), tok),
            pl.BlockSpec((_D_DOR, _N_EXP), full),
            pl.BlockSpec((_D_VEN, _N_EXP), full),
            pl.BlockSpec((_D_DOR, _N_EXP * _D_EXP), full),
            pl.BlockSpec((_D_VEN, _N_EXP * _D_EXP), full),
            pl.BlockSpec((1, _N_EXP * _D_EXP), full),
            pl.BlockSpec((_N_EXP * _D_EXP, _D_OUT), full),
            pl.BlockSpec((_N_EXP, _D_OUT), full),
            pl.BlockSpec((1, _D_OUT), full),
            pl.BlockSpec((_D_EXP, _D_DOR + _D_VEN), full),
            pl.BlockSpec((_N_EXP, _N_EXP * _D_EXP), full),
        ],
        out_specs=(
            pl.BlockSpec((block, _D_OUT), tok),
            pl.BlockSpec((block, _D_DOR), tok),
            pl.BlockSpec((block, _D_VEN), tok),
            pl.BlockSpec((block, _N_EXP), tok),
        ),
        out_shape=out_shapes,
        compiler_params=pltpu.CompilerParams(
            dimension_semantics=("arbitrary",),
        ),
    )(dorsal, ventral, gwd, gwv, w1d, w1v, b1row, w2o, b2o, borow, wf, expand)


def kernel(dorsal, ventral, gate_w, w1, b1, w2, b2, wo, bo, wfd, wfv):
    return _run(dorsal, ventral, gate_w, w1, b1, w2, b2, wo, bo, wfd, wfv)


# speedup vs baseline: 1.3088x; 1.0104x over previous
import functools

import jax
import jax.numpy as jnp
from jax.experimental import pallas as pl
from jax.experimental.pallas import tpu as pltpu

_B = 32768
_D_DOR = 128
_D_VEN = 128
_N_EXP = 8
_D_EXP = 64
_D_OUT = 64
_FB = 0.5


def _moe_kernel(d_ref, v_ref, gwd_ref, gwv_ref, w1d_ref, w1v_ref, b1_ref,
                w2o_ref, b2o_ref, bo_ref, wf_ref, exp_ref,
                assoc_ref, fbd_ref, fbv_ref, gw_ref):
    d = d_ref[...]
    v = v_ref[...]
    f32 = jnp.float32

    logits = (jnp.dot(d, gwd_ref[...], preferred_element_type=f32)
              + jnp.dot(v, gwv_ref[...], preferred_element_type=f32))

    m1 = jnp.max(logits, axis=-1, keepdims=True)
    l2 = jnp.where(logits == m1, jnp.float32(-1e30), logits)
    m2 = jnp.max(l2, axis=-1, keepdims=True)
    keep = logits >= m2

    e = jnp.where(keep, jnp.exp(logits - m1), 0.0)
    gw = e / jnp.sum(e, axis=-1, keepdims=True)
    gw_ref[...] = gw

    bf16 = jnp.bfloat16
    d16 = d.astype(bf16)
    v16 = v.astype(bf16)
    h = (jnp.dot(d16, w1d_ref[...], preferred_element_type=f32)
         + jnp.dot(v16, w1v_ref[...], preferred_element_type=f32)
         + b1_ref[...])
    h = 0.5 * h * (1.0 + jax.lax.erf(h * jnp.float32(0.7071067811865476)))

    gwx = jnp.dot(gw, exp_ref[...], preferred_element_type=f32)
    hs = (h * gwx).astype(bf16)
    assoc = (jnp.dot(hs, w2o_ref[...], preferred_element_type=f32)
             + jnp.dot(gw, b2o_ref[...], preferred_element_type=f32)
             + bo_ref[...])
    assoc_ref[...] = assoc
    fb = _FB * jnp.dot(assoc.astype(bf16), wf_ref[...], preferred_element_type=f32)
    fbd_ref[...] = fb[:, :_D_DOR]
    fbv_ref[...] = fb[:, _D_DOR:]


@functools.partial(jax.jit, static_argnames=("block",))
def _run(dorsal, ventral, gate_w, w1, b1, w2, b2, wo, bo, wfd, wfv, block=4096):
    gwT = gate_w.T
    gwd, gwv = gwT[:_D_DOR], gwT[_D_DOR:]
    w1cat = w1.transpose(2, 0, 1).reshape(_D_DOR + _D_VEN, _N_EXP * _D_EXP)
    w1cat = w1cat.astype(jnp.bfloat16)
    w1d, w1v = w1cat[:_D_DOR], w1cat[_D_DOR:]
    b1row = b1.reshape(1, _N_EXP * _D_EXP)
    w2s = w2.transpose(0, 2, 1).reshape(_N_EXP * _D_EXP, _D_EXP)
    w2o = (w2s @ wo.T).astype(jnp.bfloat16)
    b2o = b2 @ wo.T
    borow = bo.reshape(1, _D_OUT)
    wf = jnp.concatenate([wfd.T, wfv.T], axis=1).astype(jnp.bfloat16)
    expand = jnp.repeat(jnp.eye(_N_EXP, dtype=jnp.float32), _D_EXP, axis=1)

    grid = (_B // block,)
    tok = lambda i: (i, 0)
    full = lambda i: (0, 0)
    out_shapes = (
        jax.ShapeDtypeStruct((_B, _D_OUT), jnp.float32),
        jax.ShapeDtypeStruct((_B, _D_DOR), jnp.float32),
        jax.ShapeDtypeStruct((_B, _D_VEN), jnp.float32),
        jax.ShapeDtypeStruct((_B, _N_EXP), jnp.float32),
    )
    return pl.pallas_call(
        _moe_kernel,
        grid=grid,
        in_specs=[
            pl.BlockSpec((block, _D_DOR), tok),
            pl.BlockSpec((block, _D_VEN), tok),
            pl.BlockSpec((_D_DOR, _N_EXP), full),
            pl.BlockSpec((_D_VEN, _N_EXP), full),
            pl.BlockSpec((_D_DOR, _N_EXP * _D_EXP), full),
            pl.BlockSpec((_D_VEN, _N_EXP * _D_EXP), full),
            pl.BlockSpec((1, _N_EXP * _D_EXP), full),
            pl.BlockSpec((_N_EXP * _D_EXP, _D_OUT), full),
            pl.BlockSpec((_N_EXP, _D_OUT), full),
            pl.BlockSpec((1, _D_OUT), full),
            pl.BlockSpec((_D_EXP, _D_DOR + _D_VEN), full),
            pl.BlockSpec((_N_EXP, _N_EXP * _D_EXP), full),
        ],
        out_specs=(
            pl.BlockSpec((block, _D_OUT), tok),
            pl.BlockSpec((block, _D_DOR), tok),
            pl.BlockSpec((block, _D_VEN), tok),
            pl.BlockSpec((block, _N_EXP), tok),
        ),
        out_shape=out_shapes,
        compiler_params=pltpu.CompilerParams(
            dimension_semantics=("arbitrary",),
        ),
    )(dorsal, ventral, gwd, gwv, w1d, w1v, b1row, w2o, b2o, borow, wf, expand)


def kernel(dorsal, ventral, gate_w, w1, b1, w2, b2, wo, bo, wfd, wfv):
    return _run(dorsal, ventral, gate_w, w1, b1, w2, b2, wo, bo, wfd, wfv)
